# baseline (speedup 1.0000x reference)
import jax
import jax.numpy as jnp
from jax.experimental import pallas as pl
from jax.experimental.pallas import tpu as pltpu

_D = 2048
_TOTAL = 7168
_CHUNK = 1024
_NCHUNKS = _TOTAL // _CHUNK


def _topk_mask(h, k):
    imin = jnp.int32(-2147483648)
    v = jax.lax.bitcast_convert_type(h, jnp.int32)
    s = jnp.where(v >= 0, v, imin - v)

    def cnt_ge(keys, cand):
        m = (keys >= cand).astype(jnp.int16)
        w = m.shape[1]
        while w > 256:
            w //= 2
            m = m[:, :w] + m[:, w:]
        return jnp.sum(m.astype(jnp.int32), axis=1, keepdims=True)

    s_hi = (s >> 16).astype(jnp.int16)
    Th = jnp.full((h.shape[0], 1), -32768, jnp.int32)

    def body_hi(j, Th):
        cand = Th + (jnp.int32(1) << (jnp.int32(15) - j))
        ok = cnt_ge(s_hi, cand.astype(jnp.int16)) >= k
        return jnp.where(ok, cand, Th)

    Th = jax.lax.fori_loop(0, 16, body_hi, Th)
    Th16 = Th.astype(jnp.int16)

    lo16 = ((s & jnp.int32(0xFFFF)) - jnp.int32(32768)).astype(jnp.int16)
    t = jnp.where(s_hi > Th16, jnp.int16(32767),
                  jnp.where(s_hi < Th16, jnp.int16(-32768), lo16))
    Tl = jnp.full((h.shape[0], 1), -32768, jnp.int32)

    def body_lo(j, Tl):
        cand = Tl + (jnp.int32(1) << (jnp.int32(15) - j))
        ok = cnt_ge(t, cand.astype(jnp.int16)) >= k
        return jnp.where(ok, cand, Tl)

    Tl = jax.lax.fori_loop(0, 16, body_lo, Tl)

    T = (Th << 16) | (Tl + 32768)
    return jnp.where(s >= T, h, 0.0)


def _fused_body(x_ref, W_ref, b_ref, Wd_ref, bd_ref,
                zf_ref, r1_ref, r2_ref, r3_ref):
    j = pl.program_id(1)

    @pl.when(j < 14)
    def _():
        h = jnp.dot(x_ref[...], W_ref[...], preferred_element_type=jnp.float32)
        h = h + b_ref[...]

        for c in range(14):
            @pl.when(j == c)
            def _(c=c, h=h):
                zf_ref[:, c * 512:(c + 1) * 512] = h

        @pl.when(j == 1)
        def _():
            zf_ref[:, 0:1024] = _topk_mask(zf_ref[:, 0:1024], 32)

        @pl.when(j == 5)
        def _():
            zf_ref[:, 1024:3072] = _topk_mask(zf_ref[:, 1024:3072], 64)

        @pl.when(j == 13)
        def _():
            zf_ref[:, 3072:7168] = _topk_mask(zf_ref[:, 3072:7168], 128)

    for c in range(14):
        @pl.when(j == c + 14)
        def _(c=c):
            z = zf_ref[:, c * 512:(c + 1) * 512].astype(jnp.bfloat16)
            p = jnp.dot(z, Wd_ref[...], preferred_element_type=jnp.float32)
            if c == 0:
                r = p + bd_ref[...]
                r1_ref[...] = r
                r2_ref[...] = r
                r3_ref[...] = r
            elif c == 1:
                r1_ref[...] = r1_ref[...] + p
                r2_ref[...] = r2_ref[...] + p
                r3_ref[...] = r3_ref[...] + p
            elif c <= 5:
                r2_ref[...] = r2_ref[...] + p
                r3_ref[...] = r3_ref[...] + p
            else:
                r3_ref[...] = r3_ref[...] + p


def kernel(x, W1, b1, W2, b2, W3, b3, Wd, bd):
    B = x.shape[0]
    Wc = jnp.concatenate([W1, W2, W3], axis=1)
    bc = jnp.concatenate([b1, b2, b3])[None, :]
    Wd16 = Wd.astype(jnp.bfloat16)

    BM = 256
    zf, r1, r2, r3 = pl.pallas_call(
        _fused_body,
        grid=(B // BM, 28),
        in_specs=[
            pl.BlockSpec((BM, _D), lambda i, j: (i, 0)),
            pl.BlockSpec((_D, 512), lambda i, j: (0, jnp.minimum(j, 13))),
            pl.BlockSpec((1, 512), lambda i, j: (0, jnp.minimum(j, 13))),
            pl.BlockSpec((512, _D),
                         lambda i, j: (jnp.clip(j - 14, 0, 13), 0)),
            pl.BlockSpec((1, _D), lambda i, j: (0, 0)),
        ],
        out_specs=[
            pl.BlockSpec((BM, _TOTAL), lambda i, j: (i, 0)),
            pl.BlockSpec((BM, _D), lambda i, j: (i, 0)),
            pl.BlockSpec((BM, _D), lambda i, j: (i, 0)),
            pl.BlockSpec((BM, _D), lambda i, j: (i, 0)),
        ],
        out_shape=[
            jax.ShapeDtypeStruct((B, _TOTAL), jnp.float32),
            jax.ShapeDtypeStruct((B, _D), jnp.float32),
            jax.ShapeDtypeStruct((B, _D), jnp.float32),
            jax.ShapeDtypeStruct((B, _D), jnp.float32),
        ],
    )(x, Wc, bc, Wd16, bd[None, :])

    z1 = zf[:, :1024]
    z2 = zf[:, 1024:3072]
    z3 = zf[:, 3072:]
    return (r1, r2, r3, z1, z2, z3, zf)


# speedup vs baseline: 1.2275x; 1.2275x over previous
import jax
import jax.numpy as jnp
from jax.experimental import pallas as pl
from jax.experimental.pallas import tpu as pltpu

_D = 2048
_TOTAL = 7168
_CHUNK = 1024
_NCHUNKS = _TOTAL // _CHUNK


def _topk_mask(h, k):
    imin = jnp.int32(-2147483648)
    v = jax.lax.bitcast_convert_type(h, jnp.int32)
    s = jnp.where(v >= 0, v, imin - v)

    def cnt_ge(keys, cand):
        m = (keys >= cand).astype(jnp.int16)
        w = m.shape[1]
        while w > 256:
            w //= 2
            m = m[:, :w] + m[:, w:]
        return jnp.sum(m.astype(jnp.int32), axis=1, keepdims=True)

    s_hi = (s >> 16).astype(jnp.int16)
    Th = jnp.full((h.shape[0], 1), -32768, jnp.int32)

    def body_hi(j, Th):
        cand = Th + (jnp.int32(1) << (jnp.int32(15) - j))
        ok = cnt_ge(s_hi, cand.astype(jnp.int16)) >= k
        return jnp.where(ok, cand, Th)

    Th = jax.lax.fori_loop(0, 16, body_hi, Th)
    Th16 = Th.astype(jnp.int16)

    lo16 = ((s & jnp.int32(0xFFFF)) - jnp.int32(32768)).astype(jnp.int16)
    t = jnp.where(s_hi > Th16, jnp.int16(32767),
                  jnp.where(s_hi < Th16, jnp.int16(-32768), lo16))
    Tl = jnp.full((h.shape[0], 1), -32768, jnp.int32)

    def body_lo(j, Tl):
        cand = Tl + (jnp.int32(1) << (jnp.int32(15) - j))
        ok = cnt_ge(t, cand.astype(jnp.int16)) >= k
        return jnp.where(ok, cand, Tl)

    Tl = jax.lax.fori_loop(0, 16, body_lo, Tl)

    T = (Th << 16) | (Tl + 32768)
    return jnp.where(s >= T, h, 0.0)


def _enc_body(x_ref, W_ref, b_ref, zf_ref):
    nb = pl.program_id(1)
    h = jnp.dot(x_ref[...], W_ref[...], preferred_element_type=jnp.float32)
    h = h + b_ref[...]

    for c in range(_NCHUNKS):
        @pl.when(nb == c)
        def _(c=c, h=h):
            zf_ref[:, c * _CHUNK:(c + 1) * _CHUNK] = h

    @pl.when(nb == 0)
    def _():
        zf_ref[:, 0:1024] = _topk_mask(zf_ref[:, 0:1024], 32)

    @pl.when(nb == 2)
    def _():
        zf_ref[:, 1024:3072] = _topk_mask(zf_ref[:, 1024:3072], 64)

    @pl.when(nb == 6)
    def _():
        zf_ref[:, 3072:7168] = _topk_mask(zf_ref[:, 3072:7168], 128)


def _dec_body(zf_ref, Wd_ref, bd_ref, r1_ref, r2_ref, r3_ref):
    kb = pl.program_id(1)
    p = jnp.dot(zf_ref[...].astype(jnp.bfloat16), Wd_ref[...],
                preferred_element_type=jnp.float32)

    @pl.when(kb == 0)
    def _():
        r = p + bd_ref[...]
        r1_ref[...] = r
        r2_ref[...] = r
        r3_ref[...] = r

    @pl.when((kb == 1) | (kb == 2))
    def _():
        r2_ref[...] = r2_ref[...] + p
        r3_ref[...] = r3_ref[...] + p

    @pl.when(kb >= 3)
    def _():
        r3_ref[...] = r3_ref[...] + p


def kernel(x, W1, b1, W2, b2, W3, b3, Wd, bd):
    B = x.shape[0]
    Wc = jnp.concatenate([W1, W2, W3], axis=1)
    bc = jnp.concatenate([b1, b2, b3])[None, :]

    BM = 256
    zf = pl.pallas_call(
        _enc_body,
        grid=(B // BM, _NCHUNKS),
        in_specs=[
            pl.BlockSpec((BM, _D), lambda i, j: (i, 0)),
            pl.BlockSpec((_D, _CHUNK), lambda i, j: (0, j)),
            pl.BlockSpec((1, _CHUNK), lambda i, j: (0, j)),
        ],
        out_specs=pl.BlockSpec((BM, _TOTAL), lambda i, j: (i, 0)),
        out_shape=jax.ShapeDtypeStruct((B, _TOTAL), jnp.float32),
    )(x, Wc, bc)

    BM2 = 512
    Wd16 = Wd.astype(jnp.bfloat16)
    r1, r2, r3 = pl.pallas_call(
        _dec_body,
        grid=(B // BM2, _NCHUNKS),
        in_specs=[
            pl.BlockSpec((BM2, _CHUNK), lambda i, j: (i, j)),
            pl.BlockSpec((_CHUNK, _D), lambda i, j: (j, 0)),
            pl.BlockSpec((1, _D), lambda i, j: (0, 0)),
        ],
        out_specs=[pl.BlockSpec((BM2, _D), lambda i, j: (i, 0))] * 3,
        out_shape=[jax.ShapeDtypeStruct((B, _D), jnp.float32)] * 3,
    )(zf, Wd16, bd[None, :])

    z1 = zf[:, :1024]
    z2 = zf[:, 1024:3072]
    z3 = zf[:, 3072:]
    return (r1, r2, r3, z1, z2, z3, zf)
